# baseline (speedup 1.0000x reference)
import functools

import jax
import jax.numpy as jnp
from jax import lax
from jax.experimental import pallas as pl
from jax.experimental.pallas import tpu as pltpu
from jax.experimental.pallas import tpu_sc as plsc

DIM = 2048
INTER = 1024
NE = 16
T = 128
BI = 1024
NB = INTER // BI

_CDIMS = (((1,), (1,)), ((), ()))


def _swiglu_block(xb, wg_ref, wu_ref):
    wg = wg_ref[...].reshape(BI, DIM).astype(jnp.bfloat16)
    wu = wu_ref[...].reshape(BI, DIM).astype(jnp.bfloat16)
    hg = lax.dot_general(xb, wg, _CDIMS, preferred_element_type=jnp.float32)
    hu = lax.dot_general(xb, wu, _CDIMS, preferred_element_type=jnp.float32)
    return (hg * jax.nn.sigmoid(hg)) * hu


def _router_body(xb_ref, rw_ref, scores_ref):
    logits = lax.dot_general(
        xb_ref[...], rw_ref[...].astype(jnp.bfloat16), _CDIMS,
        preferred_element_type=jnp.float32,
    )
    scores_ref[...] = jax.nn.sigmoid(logits)


def _router_scores(xb, router_weight, interpret=False):
    return pl.pallas_call(
        _router_body,
        out_shape=jax.ShapeDtypeStruct((T, NE), jnp.float32),
        interpret=interpret,
    )(xb, router_weight)


def _shared_body(xb_ref, sw1g_ref, sw1u_ref, sw2_ref, out_ref):
    k = pl.program_id(0)

    @pl.when(k == 0)
    def _():
        out_ref[...] = jnp.zeros_like(out_ref)

    a = _swiglu_block(xb_ref[...], sw1g_ref, sw1u_ref)
    wd = sw2_ref[...].reshape(DIM, BI).astype(jnp.bfloat16)
    out_ref[...] += lax.dot_general(a.astype(jnp.bfloat16), wd, _CDIMS,
                                    preferred_element_type=jnp.float32)


def _shared(xb, shared_w1, shared_w2, interpret=False):
    return pl.pallas_call(
        _shared_body,
        grid=(NB,),
        in_specs=[
            pl.BlockSpec((T, DIM), lambda k: (0, 0)),
            pl.BlockSpec((BI, DIM), lambda k: (k, 0)),
            pl.BlockSpec((BI, DIM), lambda k: (k + NB, 0)),
            pl.BlockSpec((DIM, BI), lambda k: (0, k)),
        ],
        out_specs=pl.BlockSpec((T, DIM), lambda k: (0, 0)),
        out_shape=jax.ShapeDtypeStruct((T, DIM), jnp.float32),
        interpret=interpret,
    )(xb, shared_w1, shared_w1, shared_w2)


_NC = 2
_NS = 16
_NW = _NC * _NS
_TPW = T // _NW


def _gate_body(scores_hbm, rb_hbm, gt_hbm, rb_v, sc_v, g_v):
    wid = lax.axis_index("s") * _NC + lax.axis_index("c")
    base = wid * _TPW
    pltpu.sync_copy(rb_hbm, rb_v)
    pltpu.sync_copy(scores_hbm.at[pl.ds(base, _TPW)], sc_v)
    rbv = rb_v[...]
    rbs = [rbv[e] for e in range(NE)]
    lanes = lax.iota(jnp.int32, NE)
    neg = jnp.float32(-3.0e38)
    for t in range(_TPW):
        s = sc_v[t, :]
        ss = [s[e] for e in range(NE)]
        m1, i1, raw1 = neg, jnp.int32(-1), jnp.float32(0.0)
        for e in range(NE):
            sele = ss[e] + rbs[e]
            b = sele > m1
            m1 = jnp.where(b, sele, m1)
            i1 = jnp.where(b, e, i1)
            raw1 = jnp.where(b, ss[e], raw1)
        m2, i2, raw2 = neg, jnp.int32(-1), jnp.float32(0.0)
        for e in range(NE):
            sele = ss[e] + rbs[e]
            b = (sele > m2) & (i1 != e)
            m2 = jnp.where(b, sele, m2)
            i2 = jnp.where(b, e, i2)
            raw2 = jnp.where(b, ss[e], raw2)
        denom = raw1 + raw2
        top = (jnp.where(lanes == i1, raw1, jnp.float32(0.0))
               + jnp.where(lanes == i2, raw2, jnp.float32(0.0)))
        g_v[t, :] = top / denom
    pltpu.sync_copy(g_v, gt_hbm.at[pl.ds(base, _TPW)])


def _gates(scores, router_bias):
    mesh = plsc.VectorSubcoreMesh(core_axis_name="c", subcore_axis_name="s")
    f = functools.partial(
        pl.kernel,
        out_type=jax.ShapeDtypeStruct((T, NE), jnp.float32),
        mesh=mesh,
        scratch_types=[
            pltpu.VMEM((NE,), jnp.float32),
            pltpu.VMEM((_TPW, NE), jnp.float32),
            pltpu.VMEM((_TPW, NE), jnp.float32),
        ],
    )(_gate_body)
    return f(scores, router_bias)


def _experts_body(xb_ref, g_ref, shared_ref, w1g_ref, w1u_ref, w2_ref,
                  out_ref):
    e = pl.program_id(0)
    k = pl.program_id(1)

    @pl.when((e == 0) & (k == 0))
    def _():
        out_ref[...] = shared_ref[...]

    lanes = lax.broadcasted_iota(jnp.int32, (T, NE), 1)
    gcol = jnp.sum(jnp.where(lanes == e, g_ref[...], 0.0), axis=1,
                   keepdims=True)
    a = _swiglu_block(xb_ref[...], w1g_ref, w1u_ref)
    a = a * gcol
    wd = w2_ref[...].reshape(DIM, BI).astype(jnp.bfloat16)
    out_ref[...] += lax.dot_general(a.astype(jnp.bfloat16), wd, _CDIMS,
                                    preferred_element_type=jnp.float32)


def _experts(xb, g, shared_out, w1, w2, interpret=False):
    return pl.pallas_call(
        _experts_body,
        grid=(NE, NB),
        in_specs=[
            pl.BlockSpec((T, DIM), lambda e, k: (0, 0)),
            pl.BlockSpec((T, NE), lambda e, k: (0, 0)),
            pl.BlockSpec((T, DIM), lambda e, k: (0, 0)),
            pl.BlockSpec((1, BI, DIM), lambda e, k: (e, k, 0)),
            pl.BlockSpec((1, BI, DIM), lambda e, k: (e, k + NB, 0)),
            pl.BlockSpec((1, DIM, BI), lambda e, k: (e, 0, k)),
        ],
        out_specs=pl.BlockSpec((T, DIM), lambda e, k: (0, 0)),
        out_shape=jax.ShapeDtypeStruct((T, DIM), jnp.float32),
        interpret=interpret,
    )(xb, g, shared_out, w1, w1, w2)


def kernel(x, router_weight, router_bias, w1, w2, shared_w1, shared_w2):
    xb = x.astype(jnp.bfloat16)
    scores = _router_scores(xb, router_weight)
    gt = _gates(scores, router_bias)
    shared_out = _shared(xb, shared_w1, shared_w2)
    return _experts(xb, gt, shared_out, w1, w2)


# speedup vs baseline: 1.0195x; 1.0195x over previous
import functools

import jax
import jax.numpy as jnp
from jax import lax
from jax.experimental import pallas as pl
from jax.experimental.pallas import tpu as pltpu
from jax.experimental.pallas import tpu_sc as plsc

DIM = 2048
INTER = 1024
NE = 16
T = 128
BI = 1024
NB = INTER // BI

_CDIMS = (((1,), (1,)), ((), ()))


def _swiglu_block(xb, wg_ref, wu_ref):
    wg = wg_ref[...].reshape(BI, DIM).astype(jnp.bfloat16)
    wu = wu_ref[...].reshape(BI, DIM).astype(jnp.bfloat16)
    hg = lax.dot_general(xb, wg, _CDIMS, preferred_element_type=jnp.float32)
    hu = lax.dot_general(xb, wu, _CDIMS, preferred_element_type=jnp.float32)
    return (hg * jax.nn.sigmoid(hg)) * hu


def _router_body(xb_ref, rw_ref, scores_ref):
    logits = lax.dot_general(
        xb_ref[...], rw_ref[...].astype(jnp.bfloat16), _CDIMS,
        preferred_element_type=jnp.float32,
    )
    scores_ref[...] = jax.nn.sigmoid(logits)


def _router_scores(xb, router_weight, interpret=False):
    return pl.pallas_call(
        _router_body,
        out_shape=jax.ShapeDtypeStruct((T, NE), jnp.float32),
        interpret=interpret,
    )(xb, router_weight)


def _shared_body(xb_ref, sw1g_ref, sw1u_ref, sw2_ref, out_ref):
    k = pl.program_id(0)

    @pl.when(k == 0)
    def _():
        out_ref[...] = jnp.zeros_like(out_ref)

    a = _swiglu_block(xb_ref[...], sw1g_ref, sw1u_ref)
    wd = sw2_ref[...].reshape(DIM, BI).astype(jnp.bfloat16)
    out_ref[...] += lax.dot_general(a.astype(jnp.bfloat16), wd, _CDIMS,
                                    preferred_element_type=jnp.float32)


def _shared(xb, shared_w1, shared_w2, interpret=False):
    return pl.pallas_call(
        _shared_body,
        grid=(NB,),
        in_specs=[
            pl.BlockSpec((T, DIM), lambda k: (0, 0)),
            pl.BlockSpec((BI, DIM), lambda k: (k, 0)),
            pl.BlockSpec((BI, DIM), lambda k: (k + NB, 0)),
            pl.BlockSpec((DIM, BI), lambda k: (0, k)),
        ],
        out_specs=pl.BlockSpec((T, DIM), lambda k: (0, 0)),
        out_shape=jax.ShapeDtypeStruct((T, DIM), jnp.float32),
        interpret=interpret,
    )(xb, shared_w1, shared_w1, shared_w2)


_NC = 2
_NS = 16
_NW = _NC * _NS
_TPW = T // _NW


def _gate_body(scores_hbm, rb_hbm, gt_hbm, rb_v, sc_v, g_v):
    wid = lax.axis_index("s") * _NC + lax.axis_index("c")
    base = wid * _TPW
    pltpu.sync_copy(rb_hbm, rb_v)
    pltpu.sync_copy(scores_hbm.at[pl.ds(base, _TPW)], sc_v)
    rbv = rb_v[...]
    rbs = [rbv[e] for e in range(NE)]
    lanes = lax.iota(jnp.int32, NE)
    neg = jnp.float32(-3.0e38)
    for t in range(_TPW):
        s = sc_v[t, :]
        ss = [s[e] for e in range(NE)]
        m1, i1, raw1 = neg, jnp.int32(-1), jnp.float32(0.0)
        for e in range(NE):
            sele = ss[e] + rbs[e]
            b = sele > m1
            m1 = jnp.where(b, sele, m1)
            i1 = jnp.where(b, e, i1)
            raw1 = jnp.where(b, ss[e], raw1)
        m2, i2, raw2 = neg, jnp.int32(-1), jnp.float32(0.0)
        for e in range(NE):
            sele = ss[e] + rbs[e]
            b = (sele > m2) & (i1 != e)
            m2 = jnp.where(b, sele, m2)
            i2 = jnp.where(b, e, i2)
            raw2 = jnp.where(b, ss[e], raw2)
        denom = raw1 + raw2
        top = (jnp.where(lanes == i1, raw1, jnp.float32(0.0))
               + jnp.where(lanes == i2, raw2, jnp.float32(0.0)))
        g_v[t, :] = top / denom
    pltpu.sync_copy(g_v, gt_hbm.at[pl.ds(base, _TPW)])


def _gates(scores, router_bias):
    mesh = plsc.VectorSubcoreMesh(core_axis_name="c", subcore_axis_name="s")
    f = functools.partial(
        pl.kernel,
        out_type=jax.ShapeDtypeStruct((T, NE), jnp.float32),
        mesh=mesh,
        scratch_types=[
            pltpu.VMEM((NE,), jnp.float32),
            pltpu.VMEM((_TPW, NE), jnp.float32),
            pltpu.VMEM((_TPW, NE), jnp.float32),
        ],
    )(_gate_body)
    return f(scores, router_bias)


def _experts_body(xb_ref, g_ref, shared_ref, w1g_ref, w1u_ref, w2_ref,
                  out_ref):
    e = pl.program_id(0)
    k = pl.program_id(1)

    @pl.when((e == 0) & (k == 0))
    def _():
        out_ref[...] = shared_ref[...]

    lanes = lax.broadcasted_iota(jnp.int32, (T, NE), 1)
    gcol = jnp.sum(jnp.where(lanes == e, g_ref[...], 0.0), axis=1,
                   keepdims=True)
    a = _swiglu_block(xb_ref[...], w1g_ref, w1u_ref)
    a = a * gcol
    wd = w2_ref[...].reshape(DIM, BI).astype(jnp.bfloat16)
    out_ref[...] += lax.dot_general(a.astype(jnp.bfloat16), wd, _CDIMS,
                                    preferred_element_type=jnp.float32)


def _experts(xb, g, shared_out, w1, w2, interpret=False):
    return pl.pallas_call(
        _experts_body,
        grid=(NE, NB),
        in_specs=[
            pl.BlockSpec((T, DIM), lambda e, k: (0, 0)),
            pl.BlockSpec((T, NE), lambda e, k: (0, 0)),
            pl.BlockSpec((T, DIM), lambda e, k: (0, 0)),
            pl.BlockSpec((1, BI, DIM), lambda e, k: (e, k, 0)),
            pl.BlockSpec((1, BI, DIM), lambda e, k: (e, k + NB, 0)),
            pl.BlockSpec((1, DIM, BI), lambda e, k: (e, 0, k)),
        ],
        out_specs=pl.BlockSpec((T, DIM), lambda e, k: (0, 0)),
        out_shape=jax.ShapeDtypeStruct((T, DIM), jnp.float32),
        interpret=interpret,
    )(xb, g, shared_out, w1, w1, w2)


def kernel(x, router_weight, router_bias, w1, w2, shared_w1, shared_w2):
    xb = x.astype(jnp.bfloat16)
    scores = _router_scores(xb, router_weight)
    shared_out = _shared(xb, shared_w1, shared_w2)
    gt = _gates(scores, router_bias)
    return _experts(xb, gt, shared_out, w1, w2)
